# baseline (speedup 1.0000x reference)
import functools

import jax
import jax.numpy as jnp
from jax import lax
from jax.experimental import pallas as pl
from jax.experimental.pallas import tpu as pltpu
from jax.experimental.pallas import tpu_sc as plsc

NC = 2
NS = 16
NW = NC * NS
CHUNK = 128
LB = 128
VB = 4096


def _tc_repack(tableT, V, D, DP):
    def body(t_ref, o_ref):
        o_ref[:, :D] = jnp.swapaxes(t_ref[...], 0, 1)

    return pl.pallas_call(
        body,
        grid=(pl.cdiv(V, VB),),
        in_specs=[pl.BlockSpec((D, VB), lambda j: (0, j))],
        out_specs=pl.BlockSpec((VB, DP), lambda j: (j, 0)),
        out_shape=jax.ShapeDtypeStruct((V, DP), jnp.float32),
    )(tableT)


def _sc_gather(tbl384, idx, n_chunks, DP):
    mesh = plsc.VectorSubcoreMesh(core_axis_name="c", subcore_axis_name="s")
    N = NW * n_chunks * CHUNK
    per_w = n_chunks * CHUNK

    @functools.partial(
        pl.kernel,
        mesh=mesh,
        compiler_params=pltpu.CompilerParams(use_tc_tiling_on_sc=True),
        out_type=jax.ShapeDtypeStruct((N, DP), jnp.float32),
        scratch_types=[
            pltpu.VMEM((per_w,), jnp.int32),
            pltpu.VMEM((2, CHUNK, DP), jnp.float32),
            pltpu.SemaphoreType.DMA,
            pltpu.SemaphoreType.DMA,
        ],
    )
    def body(t_hbm, i_hbm, o_hbm, i_v, r_v, gsem, ssem):
        wid = lax.axis_index("s") * NC + lax.axis_index("c")
        base = wid * per_w
        pltpu.sync_copy(i_hbm.at[pl.ds(base, per_w)], i_v)

        def gather_start(g, buf):
            iv = i_v.at[pl.ds(g * CHUNK, CHUNK)]
            pltpu.async_copy(t_hbm.at[iv], r_v.at[buf], gsem)

        def gather_wait(g, buf):
            iv = i_v.at[pl.ds(g * CHUNK, CHUNK)]
            pltpu.make_async_copy(t_hbm.at[iv], r_v.at[buf], gsem).wait()

        def store_start(g):
            pltpu.async_copy(
                r_v.at[lax.rem(g, 2)],
                o_hbm.at[pl.ds(base + g * CHUNK, CHUNK)], ssem)

        def store_wait(g):
            pltpu.make_async_copy(
                r_v.at[lax.rem(g, 2)],
                o_hbm.at[pl.ds(base + g * CHUNK, CHUNK)], ssem).wait()

        gather_start(0, 0)

        def step(g, carry):
            gather_wait(g, lax.rem(g, 2))
            store_start(g)

            @pl.when(g >= 1)
            def _():
                store_wait(g - 1)

            @pl.when(g <= n_chunks - 2)
            def _():
                gather_start(g + 1, lax.rem(g + 1, 2))

            return carry

        lax.fori_loop(0, n_chunks, step, 0)
        store_wait(n_chunks - 1)

    return body(tbl384, idx)


def kernel(table, indices):
    V, D = table.shape
    B, H = indices.shape
    N = B * H
    DP = 3 * LB
    assert N % (NW * CHUNK) == 0
    n_chunks = N // (NW * CHUNK)
    idx = indices.astype(jnp.int32).reshape(N)
    tbl384 = _tc_repack(table.T, V, D, DP)
    half = N // 2
    gA = _sc_gather(tbl384, idx[:half], n_chunks // 2, DP)
    gB = _sc_gather(tbl384, idx[half:], n_chunks // 2, DP)
    outA = gA[:, :D].reshape(B // 2, H, D)
    outB = gB[:, :D].reshape(B // 2, H, D)
    return jnp.concatenate([outA, outB], axis=0)


# speedup vs baseline: 1.1301x; 1.1301x over previous
import functools

import jax
import jax.numpy as jnp
from jax import lax
from jax.experimental import pallas as pl
from jax.experimental.pallas import tpu as pltpu
from jax.experimental.pallas import tpu_sc as plsc

NC = 2
NS = 16
NW = NC * NS
CHUNK = 128
LB = 128
VB = 8192


def _tc_repack(tableT, V, D, DP):
    def body(t_ref, o_ref):
        o_ref[:, :D] = jnp.swapaxes(t_ref[...], 0, 1)

    return pl.pallas_call(
        body,
        grid=(pl.cdiv(V, VB),),
        in_specs=[pl.BlockSpec((D, VB), lambda j: (0, j))],
        out_specs=pl.BlockSpec((VB, DP), lambda j: (j, 0)),
        out_shape=jax.ShapeDtypeStruct((V, DP), jnp.float32),
    )(tableT)


def _sc_gather(tbl384, idx, n_chunks, DP):
    mesh = plsc.VectorSubcoreMesh(core_axis_name="c", subcore_axis_name="s")
    N = NW * n_chunks * CHUNK
    per_w = n_chunks * CHUNK

    @functools.partial(
        pl.kernel,
        mesh=mesh,
        compiler_params=pltpu.CompilerParams(use_tc_tiling_on_sc=True),
        out_type=jax.ShapeDtypeStruct((N, DP), jnp.float32),
        scratch_types=[
            pltpu.VMEM((per_w,), jnp.int32),
            pltpu.VMEM((2, CHUNK, DP), jnp.float32),
            pltpu.SemaphoreType.DMA,
            pltpu.SemaphoreType.DMA,
        ],
    )
    def body(t_hbm, i_hbm, o_hbm, i_v, r_v, gsem, ssem):
        wid = lax.axis_index("s") * NC + lax.axis_index("c")
        base = wid * per_w
        pltpu.sync_copy(i_hbm.at[pl.ds(base, per_w)], i_v)

        def gather_start(g, buf):
            iv = i_v.at[pl.ds(g * CHUNK, CHUNK)]
            pltpu.async_copy(t_hbm.at[iv], r_v.at[buf], gsem)

        def gather_wait(g, buf):
            iv = i_v.at[pl.ds(g * CHUNK, CHUNK)]
            pltpu.make_async_copy(t_hbm.at[iv], r_v.at[buf], gsem).wait()

        def store_start(g):
            pltpu.async_copy(
                r_v.at[lax.rem(g, 2)],
                o_hbm.at[pl.ds(base + g * CHUNK, CHUNK)], ssem)

        def store_wait(g):
            pltpu.make_async_copy(
                r_v.at[lax.rem(g, 2)],
                o_hbm.at[pl.ds(base + g * CHUNK, CHUNK)], ssem).wait()

        gather_start(0, 0)

        def step(g, carry):
            gather_wait(g, lax.rem(g, 2))
            store_start(g)

            @pl.when(g >= 1)
            def _():
                store_wait(g - 1)

            @pl.when(g <= n_chunks - 2)
            def _():
                gather_start(g + 1, lax.rem(g + 1, 2))

            return carry

        lax.fori_loop(0, n_chunks, step, 0)
        store_wait(n_chunks - 1)

    return body(tbl384, idx)


def kernel(table, indices):
    V, D = table.shape
    B, H = indices.shape
    N = B * H
    DP = 3 * LB
    assert N % (NW * CHUNK) == 0
    n_chunks = N // (NW * CHUNK)
    idx = indices.astype(jnp.int32).reshape(N)
    tbl384 = _tc_repack(table.T, V, D, DP)
    g384 = _sc_gather(tbl384, idx, n_chunks, DP)
    return g384[:, :D].reshape(B, H, D)


# speedup vs baseline: 1.1326x; 1.0022x over previous
import functools

import jax
import jax.numpy as jnp
from jax import lax
from jax.experimental import pallas as pl
from jax.experimental.pallas import tpu as pltpu
from jax.experimental.pallas import tpu_sc as plsc

NC = 2
NS = 16
NW = NC * NS
CHUNK = 128
LB = 128
VB = 10240


def _tc_repack(tableT, V, D, DP):
    def body(t_ref, o_ref):
        o_ref[:, :D] = jnp.swapaxes(t_ref[...], 0, 1)

    return pl.pallas_call(
        body,
        grid=(pl.cdiv(V, VB),),
        in_specs=[pl.BlockSpec((D, VB), lambda j: (0, j))],
        out_specs=pl.BlockSpec((VB, DP), lambda j: (j, 0)),
        out_shape=jax.ShapeDtypeStruct((V, DP), jnp.float32),
    )(tableT)


def _sc_gather(tbl384, idx, n_chunks, DP):
    mesh = plsc.VectorSubcoreMesh(core_axis_name="c", subcore_axis_name="s")
    N = NW * n_chunks * CHUNK
    per_w = n_chunks * CHUNK

    @functools.partial(
        pl.kernel,
        mesh=mesh,
        compiler_params=pltpu.CompilerParams(use_tc_tiling_on_sc=True),
        out_type=jax.ShapeDtypeStruct((N, DP), jnp.float32),
        scratch_types=[
            pltpu.VMEM((per_w,), jnp.int32),
            pltpu.VMEM((2, CHUNK, DP), jnp.float32),
            pltpu.SemaphoreType.DMA,
            pltpu.SemaphoreType.DMA,
        ],
    )
    def body(t_hbm, i_hbm, o_hbm, i_v, r_v, gsem, ssem):
        wid = lax.axis_index("s") * NC + lax.axis_index("c")
        base = wid * per_w
        pltpu.sync_copy(i_hbm.at[pl.ds(base, per_w)], i_v)

        def gather_start(g, buf):
            iv = i_v.at[pl.ds(g * CHUNK, CHUNK)]
            pltpu.async_copy(t_hbm.at[iv], r_v.at[buf], gsem)

        def gather_wait(g, buf):
            iv = i_v.at[pl.ds(g * CHUNK, CHUNK)]
            pltpu.make_async_copy(t_hbm.at[iv], r_v.at[buf], gsem).wait()

        def store_start(g):
            pltpu.async_copy(
                r_v.at[lax.rem(g, 2)],
                o_hbm.at[pl.ds(base + g * CHUNK, CHUNK)], ssem)

        def store_wait(g):
            pltpu.make_async_copy(
                r_v.at[lax.rem(g, 2)],
                o_hbm.at[pl.ds(base + g * CHUNK, CHUNK)], ssem).wait()

        gather_start(0, 0)

        def step(g, carry):
            gather_wait(g, lax.rem(g, 2))
            store_start(g)

            @pl.when(g >= 1)
            def _():
                store_wait(g - 1)

            @pl.when(g <= n_chunks - 2)
            def _():
                gather_start(g + 1, lax.rem(g + 1, 2))

            return carry

        lax.fori_loop(0, n_chunks, step, 0)
        store_wait(n_chunks - 1)

    return body(tbl384, idx)


def kernel(table, indices):
    V, D = table.shape
    B, H = indices.shape
    N = B * H
    DP = 3 * LB
    assert N % (NW * CHUNK) == 0
    n_chunks = N // (NW * CHUNK)
    idx = indices.astype(jnp.int32).reshape(N)
    tbl384 = _tc_repack(table.T, V, D, DP)
    g384 = _sc_gather(tbl384, idx, n_chunks, DP)
    return g384[:, :D].reshape(B, H, D)
